# Initial kernel scaffold; baseline (speedup 1.0000x reference)
import jax
import jax.numpy as jnp
from jax.experimental import pallas as pl


def kernel(x, edge_index, edge_weight, batch, W1, b1, W2, b2, W3, b3, Wl, bl):
    raise NotImplementedError("write your pallas kernel here")


# speedup vs baseline: 5.0997x; 5.0997x over previous
import functools

import jax
import jax.numpy as jnp
from jax import lax
from jax.experimental import pallas as pl
from jax.experimental.pallas import tpu as pltpu
from jax.experimental.pallas import tpu_sc as plsc

N = 10000
NP = 10240
E = 320000
H = 128
C = 32
G = 64

NC = 2
NS = 16
NW = NC * NS
EB = 80
EP = NW * 128 * EB
ER = EP // EB
RPT = ER // NW
BIG = 32
NBIG = RPT // BIG
SPS = NP // NS


def _deg_body(dst_hbm, ew_hbm, out_hbm, dstv, eww, zbuf, acc):
    c = lax.axis_index("c")
    s = lax.axis_index("s")
    wid = s * NC + c

    def zb(i, _):
        zbuf[pl.ds(i * 16, 16)] = jnp.zeros((16,), jnp.float32)
        return 0

    lax.fori_loop(0, SPS // 16, zb, 0)
    pltpu.sync_copy(zbuf, acc.at[pl.ds(s * SPS, SPS)])
    plsc.subcore_barrier()

    row0 = wid * RPT

    def big(bc, _):
        rbase = row0 + bc * BIG
        pltpu.sync_copy(dst_hbm.at[pl.ds(rbase, BIG)], dstv)
        pltpu.sync_copy(ew_hbm.at[pl.ds(rbase, BIG)], eww)

        def sub(j, _):
            pltpu.sync_copy(eww.at[j], acc.at[dstv.at[j]], add=True)
            return 0

        lax.fori_loop(0, BIG, sub, 0)
        return 0

    lax.fori_loop(0, NBIG, big, 0)
    plsc.subcore_barrier()
    pltpu.sync_copy(acc.at[pl.ds(s * SPS, SPS)], out_hbm.at[c, pl.ds(s * SPS, SPS)])


def _agg_body(hw_hbm, src_hbm, dst_hbm, ew_hbm, out_hbm,
              srcv, dstv, eww, rows, acc, sem):
    c = lax.axis_index("c")
    s = lax.axis_index("s")
    wid = s * NC + c

    def zr(i, _):
        for jj in range(H // 16):
            rows[i, pl.ds(jj * 16, 16)] = jnp.zeros((16,), jnp.float32)
        return 0

    lax.fori_loop(0, EB, zr, 0)

    def zc(m, _):
        pltpu.sync_copy(rows, acc.at[pl.ds(s * SPS + m * EB, EB)])
        return 0

    lax.fori_loop(0, SPS // EB, zc, 0)
    plsc.subcore_barrier()

    row0 = wid * RPT

    def big(bc, _):
        rbase = row0 + bc * BIG
        pltpu.sync_copy(src_hbm.at[pl.ds(rbase, BIG)], srcv)
        pltpu.sync_copy(dst_hbm.at[pl.ds(rbase, BIG)], dstv)

        def sub(j, _):
            pltpu.sync_copy(ew_hbm.at[rbase + j], eww)
            pltpu.async_copy(hw_hbm.at[srcv.at[j]], rows, sem).wait()

            def scale(i, _):
                w = eww[i]
                for jj in range(H // 16):
                    rows[i, pl.ds(jj * 16, 16)] = rows[i, pl.ds(jj * 16, 16)] * w
                return 0

            lax.fori_loop(0, EB, scale, 0)
            pltpu.sync_copy(rows, acc.at[dstv.at[j]], add=True)
            return 0

        lax.fori_loop(0, BIG, sub, 0)
        return 0

    lax.fori_loop(0, NBIG, big, 0)
    plsc.subcore_barrier()
    pltpu.sync_copy(acc.at[pl.ds(s * SPS, SPS)], out_hbm.at[c, pl.ds(s * SPS, SPS)])


@functools.cache
def _make_sc():
    mesh = plsc.VectorSubcoreMesh(core_axis_name="c", subcore_axis_name="s")
    deg = pl.kernel(
        _deg_body,
        out_type=jax.ShapeDtypeStruct((NC, NP), jnp.float32),
        mesh=mesh,
        scratch_types=[
            pltpu.VMEM((BIG, EB), jnp.int32),
            pltpu.VMEM((BIG, EB), jnp.float32),
            pltpu.VMEM((SPS,), jnp.float32),
            pltpu.VMEM_SHARED((NP,), jnp.float32),
        ],
    )
    agg = pl.kernel(
        _agg_body,
        out_type=jax.ShapeDtypeStruct((NC, NP, H), jnp.float32),
        mesh=mesh,
        scratch_types=[
            pltpu.VMEM((BIG, EB), jnp.int32),
            pltpu.VMEM((BIG, EB), jnp.int32),
            pltpu.VMEM((EB, 16), jnp.float32),
            pltpu.VMEM((EB, H), jnp.float32),
            pltpu.VMEM_SHARED((NP, H), jnp.float32),
            pltpu.SemaphoreType.DMA,
        ],
    )
    return deg, agg


BLK = 1024
NBLK = NP // BLK


def _dinv_from(degr):
    deg = degr[:, 0:1] + degr[:, 1:2] + 1.0
    return jnp.where(deg > 0, lax.rsqrt(deg), 0.0)


def _k1_body(xr, wr, degr, hwp_r, self_r):
    dinv = _dinv_from(degr)
    t = jnp.dot(xr[...], wr[...], preferred_element_type=jnp.float32)
    hwp = dinv * t
    hwp_r[...] = hwp
    self_r[...] = dinv * hwp


def _k2_body(partr, selfr, degr, br, wr, hwp_r, self_r):
    dinv = _dinv_from(degr)
    h = jnp.maximum(dinv * (partr[0] + partr[1]) + selfr[...] + br[...], 0.0)
    t = jnp.dot(h, wr[...], preferred_element_type=jnp.float32)
    hwp = dinv * t
    hwp_r[...] = hwp
    self_r[...] = dinv * hwp


def _k3_body(partr, selfr, degr, br, batchr, wlr, blr, out_r, sums, cnt):
    i = pl.program_id(0)

    @pl.when(i == 0)
    def _():
        sums[...] = jnp.zeros_like(sums)
        cnt[...] = jnp.zeros_like(cnt)

    dinv = _dinv_from(degr)
    h = jnp.maximum(dinv * (partr[0] + partr[1]) + selfr[...] + br[...], 0.0)
    gids = lax.broadcasted_iota(jnp.int32, (BLK, G), 1).astype(jnp.float32)
    onehot = jnp.where(batchr[...] == gids, 1.0, 0.0)
    sums[...] += lax.dot_general(onehot, h, (((0,), (0,)), ((), ())),
                                 preferred_element_type=jnp.float32)
    cnt[...] += jnp.sum(onehot, axis=0)[:, None]

    @pl.when(i == pl.num_programs(0) - 1)
    def _():
        pooled = sums[...] / jnp.maximum(cnt[...], 1.0)
        out_r[...] = jnp.dot(pooled, wlr[...],
                             preferred_element_type=jnp.float32) + blr[...]


def _make_tc(interpret=False):
    f32 = jnp.float32
    nspec = pl.BlockSpec((BLK, H), lambda i: (i, 0))
    wspec = pl.BlockSpec((H, H), lambda i: (0, 0))
    dspec = pl.BlockSpec((BLK, 2), lambda i: (i, 0))
    pspec = pl.BlockSpec((NC, BLK, H), lambda i: (0, i, 0))
    bspec = pl.BlockSpec((1, H), lambda i: (0, 0))

    k1 = pl.pallas_call(
        _k1_body,
        grid=(NBLK,),
        in_specs=[nspec, wspec, dspec],
        out_specs=[nspec, nspec],
        out_shape=[jax.ShapeDtypeStruct((NP, H), f32)] * 2,
        interpret=interpret,
    )
    k2 = pl.pallas_call(
        _k2_body,
        grid=(NBLK,),
        in_specs=[pspec, nspec, dspec, bspec, wspec],
        out_specs=[nspec, nspec],
        out_shape=[jax.ShapeDtypeStruct((NP, H), f32)] * 2,
        interpret=interpret,
    )
    k3 = pl.pallas_call(
        _k3_body,
        grid=(NBLK,),
        in_specs=[pspec, nspec, dspec, bspec,
                  pl.BlockSpec((BLK, 1), lambda i: (i, 0)),
                  pl.BlockSpec((H, C), lambda i: (0, 0)),
                  pl.BlockSpec((1, C), lambda i: (0, 0))],
        out_specs=pl.BlockSpec((G, C), lambda i: (0, 0)),
        out_shape=jax.ShapeDtypeStruct((G, C), f32),
        scratch_shapes=[pltpu.VMEM((G, H), f32), pltpu.VMEM((G, H), f32)],
        interpret=interpret,
    )
    return k1, k2, k3


_k1, _k2, _k3 = _make_tc()


def kernel(x, edge_index, edge_weight, batch, W1, b1, W2, b2, W3, b3, Wl, bl):
    f32 = jnp.float32
    ewp = jnp.pad(edge_weight, (0, EP - E))
    src = jnp.pad(edge_index[0], (0, EP - E)).reshape(ER, EB)
    dst = jnp.pad(edge_index[1], (0, EP - E)).reshape(ER, EB)
    ew2 = ewp.reshape(ER, EB)
    ew16 = jnp.broadcast_to(ewp[:, None], (EP, 16)).reshape(ER, EB, 16)

    _deg_kernel, _agg_kernel = _make_sc()
    degp = _deg_kernel(dst, ew2)
    degT = degp.T

    x_p = jnp.pad(x, ((0, NP - N), (0, 0)))
    batch_f = jnp.pad(batch.astype(f32), (0, NP - N),
                      constant_values=-1.0).reshape(NP, 1)
    b1r, b2r, b3r = b1.reshape(1, H), b2.reshape(1, H), b3.reshape(1, H)
    blr = bl.reshape(1, C)

    hw1p, self1 = _k1(x_p, W1, degT)
    p1 = _agg_kernel(hw1p, src, dst, ew16)
    hw2p, self2 = _k2(p1, self1, degT, b1r, W2)
    p2 = _agg_kernel(hw2p, src, dst, ew16)
    hw3p, self3 = _k2(p2, self2, degT, b2r, W3)
    p3 = _agg_kernel(hw3p, src, dst, ew16)
    return _k3(p3, self3, degT, b3r, batch_f, Wl, blr)


# speedup vs baseline: 7.1330x; 1.3987x over previous
import functools

import jax
import jax.numpy as jnp
from jax import lax
from jax.experimental import pallas as pl
from jax.experimental.pallas import tpu as pltpu
from jax.experimental.pallas import tpu_sc as plsc

N = 10000
NP = 10240
E = 320000
H = 128
C = 32
G = 64

NC = 2
NS = 16
NW = NC * NS
EB = 64
EP = NW * 160 * EB
ER = EP // EB
RPT = ER // NW
BIG = 32
NBIG = RPT // BIG
SPS = NP // NS


def _deg_body(sd_hbm, ew_hbm, out_hbm, idx, eww, zbuf, acc):
    c = lax.axis_index("c")
    s = lax.axis_index("s")
    wid = s * NC + c

    def zb(i, _):
        zbuf[pl.ds(i * 16, 16)] = jnp.zeros((16,), jnp.float32)
        return 0

    lax.fori_loop(0, SPS // 16, zb, 0)
    pltpu.sync_copy(zbuf, acc.at[pl.ds(s * SPS, SPS)])
    plsc.subcore_barrier()

    row0 = wid * RPT

    def big(bc, _):
        rbase = row0 + bc * BIG
        pltpu.sync_copy(sd_hbm.at[pl.ds(rbase, BIG)], idx)
        pltpu.sync_copy(ew_hbm.at[pl.ds(rbase, BIG)], eww)

        def sub(j, _):
            pltpu.sync_copy(eww.at[j], acc.at[idx.at[j, 1]], add=True)
            return 0

        lax.fori_loop(0, BIG, sub, 0)
        return 0

    lax.fori_loop(0, NBIG, big, 0)
    plsc.subcore_barrier()
    pltpu.sync_copy(acc.at[pl.ds(s * SPS, SPS)], out_hbm.at[c, pl.ds(s * SPS, SPS)])


def _agg_body(hw_hbm, sd_hbm, ew_hbm, out_hbm,
              idx, rows0, rows1, ew0, ew1, acc,
              gs0, gs1, ss0, ss1, es0, es1):
    c = lax.axis_index("c")
    s = lax.axis_index("s")
    wid = s * NC + c
    rowsb = (rows0, rows1)
    ewb = (ew0, ew1)
    gsem = (gs0, gs1)
    ssem = (ss0, ss1)
    esem = (es0, es1)

    def zr(i, _):
        for jj in range(H // 16):
            rows0[i, pl.ds(jj * 16, 16)] = jnp.zeros((16,), jnp.float32)
        return 0

    lax.fori_loop(0, EB, zr, 0)

    def zc(m, _):
        pltpu.sync_copy(rows0, acc.at[pl.ds(s * SPS + m * EB, EB)])
        return 0

    lax.fori_loop(0, SPS // EB, zc, 0)
    plsc.subcore_barrier()

    row0_ = wid * RPT

    def big(bc, _):
        rbase = row0_ + bc * BIG
        pltpu.sync_copy(sd_hbm.at[pl.ds(rbase, BIG)], idx)
        pltpu.async_copy(ew_hbm.at[rbase], ew0, es0)
        pltpu.async_copy(ew_hbm.at[rbase + 1], ew1, es1)
        pltpu.async_copy(hw_hbm.at[idx.at[0, 0]], rows0, gs0)
        for j in range(BIG):
            p = j % 2
            q = 1 - p
            pltpu.make_async_copy(hw_hbm.at[idx.at[j, 0]], rowsb[p],
                                  gsem[p]).wait()
            if j + 1 < BIG:
                if j >= 1:
                    pltpu.make_async_copy(rowsb[q], acc.at[idx.at[0, 1]],
                                          ssem[q]).wait()
                pltpu.async_copy(hw_hbm.at[idx.at[j + 1, 0]], rowsb[q], gsem[q])
            pltpu.make_async_copy(ew_hbm.at[rbase + j], ewb[p], esem[p]).wait()

            def scale(i, _):
                w = ewb[p][i]
                for jj in range(H // 16):
                    rowsb[p][i, pl.ds(jj * 16, 16)] = (
                        rowsb[p][i, pl.ds(jj * 16, 16)] * w)
                return 0

            lax.fori_loop(0, EB, scale, 0)
            if j + 2 < BIG:
                pltpu.async_copy(ew_hbm.at[rbase + j + 2], ewb[p], esem[p])
            pltpu.async_copy(rowsb[p], acc.at[idx.at[j, 1]], ssem[p], add=True)
        pltpu.make_async_copy(rows0, acc.at[idx.at[0, 1]], ss0).wait()
        pltpu.make_async_copy(rows1, acc.at[idx.at[0, 1]], ss1).wait()
        return 0

    lax.fori_loop(0, NBIG, big, 0)
    plsc.subcore_barrier()
    pltpu.sync_copy(acc.at[pl.ds(s * SPS, SPS)], out_hbm.at[c, pl.ds(s * SPS, SPS)])


@functools.cache
def _make_sc():
    mesh = plsc.VectorSubcoreMesh(core_axis_name="c", subcore_axis_name="s")
    deg = pl.kernel(
        _deg_body,
        out_type=jax.ShapeDtypeStruct((NC, NP), jnp.float32),
        mesh=mesh,
        scratch_types=[
            pltpu.VMEM((BIG, 2, EB), jnp.int32),
            pltpu.VMEM((BIG, EB), jnp.float32),
            pltpu.VMEM((SPS,), jnp.float32),
            pltpu.VMEM_SHARED((NP,), jnp.float32),
        ],
    )
    agg = pl.kernel(
        _agg_body,
        out_type=jax.ShapeDtypeStruct((NC, NP, H), jnp.float32),
        mesh=mesh,
        scratch_types=[
            pltpu.VMEM((BIG, 2, EB), jnp.int32),
            pltpu.VMEM((EB, H), jnp.float32),
            pltpu.VMEM((EB, H), jnp.float32),
            pltpu.VMEM((EB, 16), jnp.float32),
            pltpu.VMEM((EB, 16), jnp.float32),
            pltpu.VMEM_SHARED((NP, H), jnp.float32),
            pltpu.SemaphoreType.DMA,
            pltpu.SemaphoreType.DMA,
            pltpu.SemaphoreType.DMA,
            pltpu.SemaphoreType.DMA,
            pltpu.SemaphoreType.DMA,
            pltpu.SemaphoreType.DMA,
        ],
    )
    return deg, agg


BLK = 1024
NBLK = NP // BLK


def _dinv_from(degr):
    deg = degr[:, 0:1] + degr[:, 1:2] + 1.0
    return jnp.where(deg > 0, lax.rsqrt(deg), 0.0)


def _k1_body(xr, wr, degr, hwp_r, self_r):
    dinv = _dinv_from(degr)
    t = jnp.dot(xr[...], wr[...], preferred_element_type=jnp.float32)
    hwp = dinv * t
    hwp_r[...] = hwp
    self_r[...] = dinv * hwp


def _k2_body(partr, selfr, degr, br, wr, hwp_r, self_r):
    dinv = _dinv_from(degr)
    h = jnp.maximum(dinv * (partr[0] + partr[1]) + selfr[...] + br[...], 0.0)
    t = jnp.dot(h, wr[...], preferred_element_type=jnp.float32)
    hwp = dinv * t
    hwp_r[...] = hwp
    self_r[...] = dinv * hwp


def _k3_body(partr, selfr, degr, br, batchr, wlr, blr, out_r, sums, cnt):
    i = pl.program_id(0)

    @pl.when(i == 0)
    def _():
        sums[...] = jnp.zeros_like(sums)
        cnt[...] = jnp.zeros_like(cnt)

    dinv = _dinv_from(degr)
    h = jnp.maximum(dinv * (partr[0] + partr[1]) + selfr[...] + br[...], 0.0)
    gids = lax.broadcasted_iota(jnp.int32, (BLK, G), 1).astype(jnp.float32)
    onehot = jnp.where(batchr[...] == gids, 1.0, 0.0)
    sums[...] += lax.dot_general(onehot, h, (((0,), (0,)), ((), ())),
                                 preferred_element_type=jnp.float32)
    cnt[...] += jnp.sum(onehot, axis=0)[:, None]

    @pl.when(i == pl.num_programs(0) - 1)
    def _():
        pooled = sums[...] / jnp.maximum(cnt[...], 1.0)
        out_r[...] = jnp.dot(pooled, wlr[...],
                             preferred_element_type=jnp.float32) + blr[...]


def _make_tc(interpret=False):
    f32 = jnp.float32
    nspec = pl.BlockSpec((BLK, H), lambda i: (i, 0))
    wspec = pl.BlockSpec((H, H), lambda i: (0, 0))
    dspec = pl.BlockSpec((BLK, 2), lambda i: (i, 0))
    pspec = pl.BlockSpec((NC, BLK, H), lambda i: (0, i, 0))
    bspec = pl.BlockSpec((1, H), lambda i: (0, 0))

    k1 = pl.pallas_call(
        _k1_body,
        grid=(NBLK,),
        in_specs=[nspec, wspec, dspec],
        out_specs=[nspec, nspec],
        out_shape=[jax.ShapeDtypeStruct((NP, H), f32)] * 2,
        interpret=interpret,
    )
    k2 = pl.pallas_call(
        _k2_body,
        grid=(NBLK,),
        in_specs=[pspec, nspec, dspec, bspec, wspec],
        out_specs=[nspec, nspec],
        out_shape=[jax.ShapeDtypeStruct((NP, H), f32)] * 2,
        interpret=interpret,
    )
    k3 = pl.pallas_call(
        _k3_body,
        grid=(NBLK,),
        in_specs=[pspec, nspec, dspec, bspec,
                  pl.BlockSpec((BLK, 1), lambda i: (i, 0)),
                  pl.BlockSpec((H, C), lambda i: (0, 0)),
                  pl.BlockSpec((1, C), lambda i: (0, 0))],
        out_specs=pl.BlockSpec((G, C), lambda i: (0, 0)),
        out_shape=jax.ShapeDtypeStruct((G, C), f32),
        scratch_shapes=[pltpu.VMEM((G, H), f32), pltpu.VMEM((G, H), f32)],
        interpret=interpret,
    )
    return k1, k2, k3


_k1, _k2, _k3 = _make_tc()


def kernel(x, edge_index, edge_weight, batch, W1, b1, W2, b2, W3, b3, Wl, bl):
    f32 = jnp.float32
    ewp = jnp.pad(edge_weight, (0, EP - E))
    src = jnp.pad(edge_index[0], (0, EP - E)).reshape(ER, EB)
    dst = jnp.pad(edge_index[1], (0, EP - E)).reshape(ER, EB)
    srcdst = jnp.stack([src, dst], axis=1)
    ew2 = ewp.reshape(ER, EB)
    ew16 = jnp.broadcast_to(ewp[:, None], (EP, 16)).reshape(ER, EB, 16)

    _deg_kernel, _agg_kernel = _make_sc()
    degp = _deg_kernel(srcdst, ew2)
    degT = degp.T

    x_p = jnp.pad(x, ((0, NP - N), (0, 0)))
    batch_f = jnp.pad(batch.astype(f32), (0, NP - N),
                      constant_values=-1.0).reshape(NP, 1)
    b1r, b2r, b3r = b1.reshape(1, H), b2.reshape(1, H), b3.reshape(1, H)
    blr = bl.reshape(1, C)

    hw1p, self1 = _k1(x_p, W1, degT)
    p1 = _agg_kernel(hw1p, srcdst, ew16)
    hw2p, self2 = _k2(p1, self1, degT, b1r, W2)
    p2 = _agg_kernel(hw2p, srcdst, ew16)
    hw3p, self3 = _k2(p2, self2, degT, b2r, W3)
    p3 = _agg_kernel(hw3p, srcdst, ew16)
    return _k3(p3, self3, degT, b3r, batch_f, Wl, blr)


# speedup vs baseline: 7.6238x; 1.0688x over previous
import functools

import jax
import jax.numpy as jnp
from jax import lax
from jax.experimental import pallas as pl
from jax.experimental.pallas import tpu as pltpu
from jax.experimental.pallas import tpu_sc as plsc

N = 10000
NP = 10240
E = 320000
H = 128
C = 32
G = 64

NC = 2
NS = 16
NW = NC * NS
EB = 64
EP = NW * 160 * EB
ER = EP // EB
RPT = ER // NW
BIG = 32
NBIG = RPT // BIG
SPS = NP // NS


def _deg_body(sd_hbm, ew_hbm, out_hbm, idx, eww, zbuf, acc):
    c = lax.axis_index("c")
    s = lax.axis_index("s")
    wid = s * NC + c

    def zb(i, _):
        zbuf[pl.ds(i * 16, 16)] = jnp.zeros((16,), jnp.float32)
        return 0

    lax.fori_loop(0, SPS // 16, zb, 0)
    pltpu.sync_copy(zbuf, acc.at[pl.ds(s * SPS, SPS)])
    plsc.subcore_barrier()

    row0 = wid * RPT

    def big(bc, _):
        rbase = row0 + bc * BIG
        pltpu.sync_copy(sd_hbm.at[pl.ds(rbase, BIG)], idx)
        pltpu.sync_copy(ew_hbm.at[pl.ds(rbase, BIG)], eww)

        def sub(j, _):
            pltpu.sync_copy(eww.at[j], acc.at[idx.at[j, 1]], add=True)
            return 0

        lax.fori_loop(0, BIG, sub, 0)
        return 0

    lax.fori_loop(0, NBIG, big, 0)
    plsc.subcore_barrier()
    pltpu.sync_copy(acc.at[pl.ds(s * SPS, SPS)], out_hbm.at[c, pl.ds(s * SPS, SPS)])


def _agg_body(hw_hbm, sd_hbm, ew_hbm, out_hbm,
              idx, rows0, rows1, ew0, ew1, acc,
              gs0, gs1, ss0, ss1, es0, es1):
    c = lax.axis_index("c")
    s = lax.axis_index("s")
    wid = s * NC + c
    rowsb = (rows0, rows1)
    ewb = (ew0, ew1)
    gsem = (gs0, gs1)
    ssem = (ss0, ss1)
    esem = (es0, es1)

    def zr(i, _):
        for jj in range(H // 16):
            rows0[i, pl.ds(jj * 16, 16)] = jnp.zeros((16,), jnp.float32)
        return 0

    lax.fori_loop(0, EB, zr, 0)

    def zc(m, _):
        pltpu.sync_copy(rows0, acc.at[pl.ds(s * SPS + m * EB, EB)])
        return 0

    lax.fori_loop(0, SPS // EB, zc, 0)
    plsc.subcore_barrier()

    row0_ = wid * RPT

    def big(bc, _):
        rbase = row0_ + bc * BIG
        pltpu.sync_copy(sd_hbm.at[pl.ds(rbase, BIG)], idx)
        pltpu.async_copy(ew_hbm.at[rbase], ew0, es0)
        pltpu.async_copy(ew_hbm.at[rbase + 1], ew1, es1)
        pltpu.async_copy(hw_hbm.at[idx.at[0, 0]], rows0, gs0)
        for j in range(BIG):
            p = j % 2
            q = 1 - p
            if j + 1 < BIG:
                if j >= 1:
                    pltpu.make_async_copy(rowsb[q], acc.at[idx.at[0, 1]],
                                          ssem[q]).wait()
                pltpu.async_copy(hw_hbm.at[idx.at[j + 1, 0]], rowsb[q], gsem[q])
            pltpu.make_async_copy(hw_hbm.at[idx.at[j, 0]], rowsb[p],
                                  gsem[p]).wait()
            pltpu.make_async_copy(ew_hbm.at[rbase + j], ewb[p], esem[p]).wait()

            def scale(i, _):
                w = ewb[p][i]
                for jj in range(H // 16):
                    rowsb[p][i, pl.ds(jj * 16, 16)] = (
                        rowsb[p][i, pl.ds(jj * 16, 16)] * w)
                return 0

            lax.fori_loop(0, EB, scale, 0)
            if j + 2 < BIG:
                pltpu.async_copy(ew_hbm.at[rbase + j + 2], ewb[p], esem[p])
            pltpu.async_copy(rowsb[p], acc.at[idx.at[j, 1]], ssem[p], add=True)
        pltpu.make_async_copy(rows0, acc.at[idx.at[0, 1]], ss0).wait()
        pltpu.make_async_copy(rows1, acc.at[idx.at[0, 1]], ss1).wait()
        return 0

    lax.fori_loop(0, NBIG, big, 0)
    plsc.subcore_barrier()
    pltpu.sync_copy(acc.at[pl.ds(s * SPS, SPS)], out_hbm.at[c, pl.ds(s * SPS, SPS)])


@functools.cache
def _make_sc():
    mesh = plsc.VectorSubcoreMesh(core_axis_name="c", subcore_axis_name="s")
    deg = pl.kernel(
        _deg_body,
        out_type=jax.ShapeDtypeStruct((NC, NP), jnp.float32),
        mesh=mesh,
        scratch_types=[
            pltpu.VMEM((BIG, 2, EB), jnp.int32),
            pltpu.VMEM((BIG, EB), jnp.float32),
            pltpu.VMEM((SPS,), jnp.float32),
            pltpu.VMEM_SHARED((NP,), jnp.float32),
        ],
    )
    agg = pl.kernel(
        _agg_body,
        out_type=jax.ShapeDtypeStruct((NC, NP, H), jnp.float32),
        mesh=mesh,
        scratch_types=[
            pltpu.VMEM((BIG, 2, EB), jnp.int32),
            pltpu.VMEM((EB, H), jnp.float32),
            pltpu.VMEM((EB, H), jnp.float32),
            pltpu.VMEM((EB, 16), jnp.float32),
            pltpu.VMEM((EB, 16), jnp.float32),
            pltpu.VMEM_SHARED((NP, H), jnp.float32),
            pltpu.SemaphoreType.DMA,
            pltpu.SemaphoreType.DMA,
            pltpu.SemaphoreType.DMA,
            pltpu.SemaphoreType.DMA,
            pltpu.SemaphoreType.DMA,
            pltpu.SemaphoreType.DMA,
        ],
    )
    return deg, agg


BLK = 1024
NBLK = NP // BLK


def _dinv_from(degr):
    deg = degr[:, 0:1] + degr[:, 1:2] + 1.0
    return jnp.where(deg > 0, lax.rsqrt(deg), 0.0)


def _k1_body(xr, wr, degr, hwp_r, self_r):
    dinv = _dinv_from(degr)
    t = jnp.dot(xr[...], wr[...], preferred_element_type=jnp.float32)
    hwp = dinv * t
    hwp_r[...] = hwp
    self_r[...] = dinv * hwp


def _k2_body(partr, selfr, degr, br, wr, hwp_r, self_r):
    dinv = _dinv_from(degr)
    h = jnp.maximum(dinv * (partr[0] + partr[1]) + selfr[...] + br[...], 0.0)
    t = jnp.dot(h, wr[...], preferred_element_type=jnp.float32)
    hwp = dinv * t
    hwp_r[...] = hwp
    self_r[...] = dinv * hwp


def _k3_body(partr, selfr, degr, br, batchr, wlr, blr, out_r, sums, cnt):
    i = pl.program_id(0)

    @pl.when(i == 0)
    def _():
        sums[...] = jnp.zeros_like(sums)
        cnt[...] = jnp.zeros_like(cnt)

    dinv = _dinv_from(degr)
    h = jnp.maximum(dinv * (partr[0] + partr[1]) + selfr[...] + br[...], 0.0)
    gids = lax.broadcasted_iota(jnp.int32, (BLK, G), 1).astype(jnp.float32)
    onehot = jnp.where(batchr[...] == gids, 1.0, 0.0)
    sums[...] += lax.dot_general(onehot, h, (((0,), (0,)), ((), ())),
                                 preferred_element_type=jnp.float32)
    cnt[...] += jnp.sum(onehot, axis=0)[:, None]

    @pl.when(i == pl.num_programs(0) - 1)
    def _():
        pooled = sums[...] / jnp.maximum(cnt[...], 1.0)
        out_r[...] = jnp.dot(pooled, wlr[...],
                             preferred_element_type=jnp.float32) + blr[...]


def _make_tc(interpret=False):
    f32 = jnp.float32
    nspec = pl.BlockSpec((BLK, H), lambda i: (i, 0))
    wspec = pl.BlockSpec((H, H), lambda i: (0, 0))
    dspec = pl.BlockSpec((BLK, 2), lambda i: (i, 0))
    pspec = pl.BlockSpec((NC, BLK, H), lambda i: (0, i, 0))
    bspec = pl.BlockSpec((1, H), lambda i: (0, 0))

    k1 = pl.pallas_call(
        _k1_body,
        grid=(NBLK,),
        in_specs=[nspec, wspec, dspec],
        out_specs=[nspec, nspec],
        out_shape=[jax.ShapeDtypeStruct((NP, H), f32)] * 2,
        interpret=interpret,
    )
    k2 = pl.pallas_call(
        _k2_body,
        grid=(NBLK,),
        in_specs=[pspec, nspec, dspec, bspec, wspec],
        out_specs=[nspec, nspec],
        out_shape=[jax.ShapeDtypeStruct((NP, H), f32)] * 2,
        interpret=interpret,
    )
    k3 = pl.pallas_call(
        _k3_body,
        grid=(NBLK,),
        in_specs=[pspec, nspec, dspec, bspec,
                  pl.BlockSpec((BLK, 1), lambda i: (i, 0)),
                  pl.BlockSpec((H, C), lambda i: (0, 0)),
                  pl.BlockSpec((1, C), lambda i: (0, 0))],
        out_specs=pl.BlockSpec((G, C), lambda i: (0, 0)),
        out_shape=jax.ShapeDtypeStruct((G, C), f32),
        scratch_shapes=[pltpu.VMEM((G, H), f32), pltpu.VMEM((G, H), f32)],
        interpret=interpret,
    )
    return k1, k2, k3


_k1, _k2, _k3 = _make_tc()


def kernel(x, edge_index, edge_weight, batch, W1, b1, W2, b2, W3, b3, Wl, bl):
    f32 = jnp.float32
    ewp = jnp.pad(edge_weight, (0, EP - E))
    src = jnp.pad(edge_index[0], (0, EP - E)).reshape(ER, EB)
    dst = jnp.pad(edge_index[1], (0, EP - E)).reshape(ER, EB)
    srcdst = jnp.stack([src, dst], axis=1)
    ew2 = ewp.reshape(ER, EB)
    ew16 = jnp.broadcast_to(ewp[:, None], (EP, 16)).reshape(ER, EB, 16)

    _deg_kernel, _agg_kernel = _make_sc()
    degp = _deg_kernel(srcdst, ew2)
    degT = degp.T

    x_p = jnp.pad(x, ((0, NP - N), (0, 0)))
    batch_f = jnp.pad(batch.astype(f32), (0, NP - N),
                      constant_values=-1.0).reshape(NP, 1)
    b1r, b2r, b3r = b1.reshape(1, H), b2.reshape(1, H), b3.reshape(1, H)
    blr = bl.reshape(1, C)

    hw1p, self1 = _k1(x_p, W1, degT)
    p1 = _agg_kernel(hw1p, srcdst, ew16)
    hw2p, self2 = _k2(p1, self1, degT, b1r, W2)
    p2 = _agg_kernel(hw2p, srcdst, ew16)
    hw3p, self3 = _k2(p2, self2, degT, b2r, W3)
    p3 = _agg_kernel(hw3p, srcdst, ew16)
    return _k3(p3, self3, degT, b3r, batch_f, Wl, blr)
